# baseline (speedup 1.0000x reference)
import functools

import jax
import jax.numpy as jnp
from jax import lax
from jax.experimental import pallas as pl
from jax.experimental.pallas import tpu as pltpu
from jax.experimental.pallas import tpu_sc as plsc

V = 1_000_000
D = 64
B_ROWS = 16384
B_COLS = 26
NB = B_ROWS * B_COLS
NC = 2
NS = 16
NW = NC * NS
PER_W = NB // NW
NCHK = 32
CITEMS = 16 * B_COLS


def kernel(input_, weight):
    idx_perm = (
        input_.reshape(NW, NCHK, 16, B_COLS)
        .transpose(0, 1, 3, 2)
        .reshape(NW, PER_W)
    )
    mesh = plsc.VectorSubcoreMesh(core_axis_name="c", subcore_axis_name="s")

    @functools.partial(
        pl.kernel,
        mesh=mesh,
        out_type=jax.ShapeDtypeStruct((B_COLS, 8, 128, 8, 128), jnp.float32),
        scratch_types=[
            pltpu.VMEM((PER_W,), jnp.int32),
            pltpu.VMEM((2, CITEMS, D), jnp.float32),
            pltpu.VMEM((1, B_COLS, 8, 8, 17), jnp.float32),
            pltpu.SemaphoreType.DMA,
            pltpu.SemaphoreType.DMA,
            pltpu.SemaphoreType.DMA,
            pltpu.SemaphoreType.DMA,
        ],
        compiler_params=pltpu.CompilerParams(
            use_tc_tiling_on_sc=False, needs_layout_passes=False),
    )
    def sc_gather(table_hbm, idx_hbm, out_hbm, idx_v, stag, stout,
                  gsem0, gsem1, osem0, osem1):
        wid = lax.axis_index("s") * NC + lax.axis_index("c")
        pltpu.sync_copy(idx_hbm.at[wid], idx_v)
        gsems = (gsem0, gsem1)
        osems = (osem0, osem1)
        pieces = ((0, 128), (128, 128), (256, 128), (384, 32))

        def fire_gather(c, gslot):
            for off, ln in pieces:
                pltpu.async_copy(
                    table_hbm.at[idx_v.at[pl.ds(c * CITEMS + off, ln)]],
                    stag.at[gslot, pl.ds(off, ln)], gsems[gslot])

        def wait_gather(gslot):
            for off, ln in pieces:
                pltpu.make_async_copy(
                    table_hbm.at[pl.ds(0, ln)],
                    stag.at[gslot, pl.ds(off, ln)], gsems[gslot]).wait()

        def out_dst(c):
            ct = 4 * wid + c // 8
            c0 = (c % 8) * 16
            return out_hbm.at[:, :, ct, :, pl.ds(c0, 16)]

        def fire_out(c, oslot):
            pltpu.async_copy(
                stout.at[oslot, :, :, :, pl.ds(0, 16)], out_dst(c),
                osems[oslot])

        def wait_out(oslot):
            pltpu.make_async_copy(
                stout.at[oslot, :, :, :, pl.ds(0, 16)],
                out_hbm.at[:, :, 0, :, pl.ds(0, 16)], osems[oslot]).wait()

        iota = lax.iota(jnp.int32, 16)
        dtvs = [(iota + 16 * k) >> 3 for k in range(4)]
        rv = iota & 7

        def transpose(gslot, oslot):
            dst = stout.at[oslot]

            @plsc.parallel_loop(0, B_COLS, unroll=4)
            def _body(b2):
                b2v = jnp.full((16,), b2, jnp.int32)
                for j in range(16):
                    jv = jnp.full((16,), j, jnp.int32)
                    s = b2 * 16 + j
                    for k in range(4):
                        vals = stag[gslot, s, pl.ds(16 * k, 16)]
                        plsc.store_scatter(dst, [b2v, dtvs[k], rv, jv], vals)

        fire_gather(0, 0)

        def chunk_step(c, gslot, oslot):
            @pl.when(c + 1 < NCHK)
            def _():
                fire_gather(c + 1, 1 - gslot)
            wait_gather(gslot)

            @pl.when(c >= 1)
            def _():
                wait_out(0)
            transpose(gslot, oslot)
            fire_out(c, oslot)

        def body(i, carry):
            c = 2 * i
            chunk_step(c, 0, 0)
            chunk_step(c + 1, 1, 0)
            return carry

        lax.fori_loop(0, NCHK // 2, body, 0)
        wait_out(0)

    out5 = sc_gather(weight, idx_perm)
    return out5.transpose(2, 4, 0, 1, 3).reshape(B_ROWS, B_COLS, D)


# speedup vs baseline: 1.0455x; 1.0455x over previous
import functools

import jax
import jax.numpy as jnp
from jax import lax
from jax.experimental import pallas as pl
from jax.experimental.pallas import tpu as pltpu
from jax.experimental.pallas import tpu_sc as plsc

V = 1_000_000
D = 64
B_ROWS = 16384
B_COLS = 26
NB = B_ROWS * B_COLS
NC = 2
NS = 16
NW = NC * NS
PER_W = NB // NW
NCHK = 32
CITEMS = 16 * B_COLS


def kernel(input_, weight):
    idx_perm = (
        input_.reshape(NW, NCHK, 16, B_COLS)
        .transpose(0, 1, 3, 2)
        .reshape(NW, PER_W)
    )
    mesh = plsc.VectorSubcoreMesh(core_axis_name="c", subcore_axis_name="s")

    @functools.partial(
        pl.kernel,
        mesh=mesh,
        out_type=jax.ShapeDtypeStruct((B_COLS, 8, 128, 8, 128), jnp.float32),
        scratch_types=[
            pltpu.VMEM((PER_W,), jnp.int32),
            pltpu.VMEM((2, CITEMS, D), jnp.float32),
            pltpu.VMEM((1, B_COLS, 8, 8, 17), jnp.float32),
            pltpu.SemaphoreType.DMA,
            pltpu.SemaphoreType.DMA,
            pltpu.SemaphoreType.DMA,
            pltpu.SemaphoreType.DMA,
        ],
        compiler_params=pltpu.CompilerParams(
            use_tc_tiling_on_sc=False, needs_layout_passes=False),
    )
    def sc_gather(table_hbm, idx_hbm, out_hbm, idx_v, stag, stout,
                  gsem0, gsem1, osem0, osem1):
        wid = lax.axis_index("s") * NC + lax.axis_index("c")
        pltpu.sync_copy(idx_hbm.at[wid], idx_v)
        gsems = (gsem0, gsem1)
        osems = (osem0, osem1)
        pieces = ((0, 128), (128, 128), (256, 128), (384, 32))

        def fire_gather(c, gslot):
            for off, ln in pieces:
                pltpu.async_copy(
                    table_hbm.at[idx_v.at[pl.ds(c * CITEMS + off, ln)]],
                    stag.at[gslot, pl.ds(off, ln)], gsems[gslot])

        def wait_gather(gslot):
            for off, ln in pieces:
                pltpu.make_async_copy(
                    table_hbm.at[pl.ds(0, ln)],
                    stag.at[gslot, pl.ds(off, ln)], gsems[gslot]).wait()

        def out_dst(c):
            ct = 4 * wid + c // 8
            c0 = (c % 8) * 16
            return out_hbm.at[:, :, ct, :, pl.ds(c0, 16)]

        def fire_out(c, oslot):
            pltpu.async_copy(
                stout.at[oslot, :, :, :, pl.ds(0, 16)], out_dst(c),
                osems[oslot])

        def wait_out(oslot):
            pltpu.make_async_copy(
                stout.at[oslot, :, :, :, pl.ds(0, 16)],
                out_hbm.at[:, :, 0, :, pl.ds(0, 16)], osems[oslot]).wait()

        iota = lax.iota(jnp.int32, 16)
        dtvs = [(iota + 16 * k) >> 3 for k in range(4)]
        rv = iota & 7

        def transpose(gslot, oslot):
            dst = stout.at[oslot]

            @plsc.parallel_loop(0, B_COLS, unroll=2)
            def _body(b2):
                b2v = jnp.full((16,), b2, jnp.int32)
                for j in range(16):
                    jv = jnp.full((16,), j, jnp.int32)
                    s = b2 * 16 + j
                    for k in range(4):
                        vals = stag[gslot, s, pl.ds(16 * k, 16)]
                        plsc.store_scatter(dst, [b2v, dtvs[k], rv, jv], vals)

        fire_gather(0, 0)

        def chunk_step(c, gslot, oslot):
            @pl.when(c + 1 < NCHK)
            def _():
                fire_gather(c + 1, 1 - gslot)
            wait_gather(gslot)

            @pl.when(c >= 1)
            def _():
                wait_out(0)
            transpose(gslot, oslot)
            fire_out(c, oslot)

        def body(i, carry):
            c = 2 * i
            chunk_step(c, 0, 0)
            chunk_step(c + 1, 1, 0)
            return carry

        lax.fori_loop(0, NCHK // 2, body, 0)
        wait_out(0)

    out5 = sc_gather(weight, idx_perm)
    return out5.transpose(2, 4, 0, 1, 3).reshape(B_ROWS, B_COLS, D)
